# baseline (speedup 1.0000x reference)
import functools
import numpy as np
import jax
import jax.numpy as jnp
from jax import lax
from jax.experimental import pallas as pl
from jax.experimental.pallas import tpu as pltpu
from jax.experimental.pallas import tpu_sc as plsc

_N = 10000
_E = 320000


def _tc1_body(x_ref, w_ref, g_ref, gd_ref, src_ref, dst_ref):
    h = jnp.dot(x_ref[...], w_ref[...], preferred_element_type=jnp.float32)
    src_ref[...] = jnp.dot(h, g_ref[...], preferred_element_type=jnp.float32)
    dst_ref[...] = jnp.dot(h, gd_ref[...], preferred_element_type=jnp.float32)


def _tc2_body(a0_ref, a1_ref, s_ref, r_ref, b_ref, w2_ref, g2_ref, gd2_ref,
              src2_ref, dst2_ref):
    acc = a0_ref[...] + a1_ref[...]
    numer = jnp.dot(acc, s_ref[...], preferred_element_type=jnp.float32)
    denom = jnp.dot(acc, r_ref[...], preferred_element_type=jnp.float32)
    o = numer / (denom + 1e-16) + b_ref[...]
    o = jnp.where(o > 0, o, jnp.exp(o) - 1.0)
    h2 = jnp.dot(o, w2_ref[...], preferred_element_type=jnp.float32)
    src2_ref[...] = jnp.dot(
        h2, g2_ref[...],
        preferred_element_type=jnp.float32).astype(src2_ref.dtype)
    dst2_ref[...] = jnp.dot(h2, gd2_ref[...], preferred_element_type=jnp.float32)


def _tc3_body(a0_ref, a1_ref, s_ref, r_ref, b_ref, out_ref):
    acc = a0_ref[...] + a1_ref[...]
    numer = jnp.dot(acc, s_ref[...], preferred_element_type=jnp.float32)
    denom = jnp.dot(acc, r_ref[...], preferred_element_type=jnp.float32)
    z = numer / (denom + 1e-16) + b_ref[...]
    m = jnp.max(z, axis=1, keepdims=True)
    out_ref[...] = z - (m + jnp.log(jnp.sum(jnp.exp(z - m), axis=1,
                                            keepdims=True)))


def _make_sc_edge_kernel(n, e, dh, heads, B, idxc, bf16_src=False):
    drep = 16
    row = dh + drep
    if bf16_src:
        srow, sdt = (dh + 32) // 2, jnp.int32
    else:
        srow, sdt = dh + 16, jnp.float32
    info = plsc.get_sparse_core_info()
    nc, ns = info.num_cores, info.num_subcores
    nw = nc * ns
    epw = e // nw
    nchunks = epw // B
    ngroups = nchunks // idxc
    CH = 40
    nch = n // CH
    cpt = nch // ns
    rem = nch - cpt * ns
    assert epw * nw == e and nchunks * B == epw and nch * CH == n
    assert ngroups * idxc == nchunks and (idxc % 2 == 0 or ngroups == 1)
    assert not bf16_src or heads == 1
    mesh = plsc.VectorSubcoreMesh(core_axis_name="c", subcore_axis_name="s")

    @functools.partial(
        pl.kernel,
        out_type=jax.ShapeDtypeStruct((nc, n, row), jnp.float32),
        mesh=mesh,
        compiler_params=pltpu.CompilerParams(use_tc_tiling_on_sc=False),
        scratch_types=[
            pltpu.VMEM((idxc, B), jnp.int32),
            pltpu.VMEM((idxc, B), jnp.int32),
            pltpu.VMEM((B, srow), sdt),
            pltpu.VMEM((B, srow), sdt),
            pltpu.VMEM((B, drep), jnp.float32),
            pltpu.VMEM((B, drep), jnp.float32),
            pltpu.VMEM((B, row), jnp.float32),
            pltpu.VMEM((B, row), jnp.float32),
            pltpu.VMEM((CH, row), jnp.float32),
            pltpu.VMEM_SHARED((n, row), jnp.float32),
            pltpu.SemaphoreType.DMA,
            pltpu.SemaphoreType.DMA,
        ],
    )
    def k(src_hbm, dst_hbm, stab_hbm, dtab_hbm, out_hbm,
          sidx_all, didx_all,
          rows0, rows1, drows0, drows1, orows0, orows1,
          zbuf, accum, gsem0, gsem1):
        rows = (rows0, rows1)
        drows = (drows0, drows1)
        orows = (orows0, orows1)
        gsem = (gsem0, gsem1)
        cid = lax.axis_index("c")
        sid = lax.axis_index("s")
        wid = sid * nc + cid
        zero = jnp.zeros((16,), jnp.float32)

        def zrow(i, carry):
            for t in range(row // 16):
                zbuf[i, pl.ds(16 * t, 16)] = zero
            return carry

        lax.fori_loop(0, CH, zrow, 0)

        def zcopy(c0):
            pltpu.async_copy(zbuf, accum.at[pl.ds(c0 * CH, CH)], gsem1)

        def zwait(c0):
            pltpu.make_async_copy(zbuf, accum.at[pl.ds(c0 * CH, CH)],
                                  gsem1).wait()

        for t in range(cpt):
            zcopy(sid * cpt + t)
        if rem:
            @pl.when(sid < rem)
            def _zero_extra():
                zcopy(cpt * ns + sid)
        for t in range(cpt):
            zwait(sid * cpt + t)
        if rem:
            @pl.when(sid < rem)
            def _zero_wait_extra():
                zwait(cpt * ns + sid)
        plsc.subcore_barrier()

        def fire(lj, b):
            pltpu.async_copy(stab_hbm.at[sidx_all.at[lj]], rows[b], gsem[b])
            pltpu.async_copy(dtab_hbm.at[didx_all.at[lj]], drows[b], gsem[b])

        def wait_gather(b):
            pltpu.make_async_copy(stab_hbm.at[sidx_all.at[0]], rows[b],
                                  gsem[b]).wait()
            pltpu.make_async_copy(dtab_hbm.at[didx_all.at[0]], drows[b],
                                  gsem[b]).wait()

        def compute(b):
            ro, dro, oro = rows[b], drows[b], orows[b]

            def unpack2(w):
                lo = lax.convert_element_type(
                    lax.shift_right_arithmetic(lax.shift_left(w, 16), 16),
                    jnp.float32)
                hi = lax.convert_element_type(
                    lax.shift_right_arithmetic(w, 16), jnp.float32)
                return lo, hi

            @plsc.parallel_loop(0, B, unroll=4)
            def _edges(ei):
                if bf16_src:
                    vas_q, _ = unpack2(ro[ei, pl.ds(dh // 2, 16)])
                    vas = vas_q * jnp.float32(1.0 / 512.0)
                else:
                    vas = ro[ei, pl.ds(dh, 16)]
                vad = dro[ei, pl.ds(0, 16)]
                a = vas + vad
                a = jnp.where(a >= 0, a, 0.2 * a)
                p = jnp.exp(a)
                oro[ei, pl.ds(dh, 16)] = p

                def scale(kk):
                    if heads == 1:
                        pidx = jnp.zeros((16,), jnp.int32)
                    else:
                        pidx = 2 * kk + lax.shift_right_logical(
                            lax.iota(jnp.int32, 16), 3)
                    return p.at[pidx].get(mode='promise_in_bounds')

                if bf16_src:
                    svq = scale(0) * jnp.float32(1.0 / 512.0)
                    for j in range(dh // 32):
                        ha, hb = unpack2(ro[ei, pl.ds(16 * j, 16)])
                        oro[ei, pl.ds(32 * j, 16)] = ha * svq
                        oro[ei, pl.ds(32 * j + 16, 16)] = hb * svq
                else:
                    for kk in range(dh // 16):
                        oro[ei, pl.ds(16 * kk, 16)] = (
                            ro[ei, pl.ds(16 * kk, 16)] * scale(kk))

        def pair(jj, carry):
            j0 = jj * 2
            for b in range(2):
                lj = j0 + b
                nb = 1 - b

                @pl.when(lj < idxc)
                def _section():
                    @pl.when(lj + 1 < idxc)
                    def _fire_next():
                        fire(lj + 1, nb)

                    wait_gather(b)
                    compute(b)
                    pltpu.sync_copy(orows[b], accum.at[didx_all.at[lj]],
                                    add=True)
            return carry

        for g in range(ngroups):
            pltpu.sync_copy(src_hbm.at[wid, g], sidx_all)
            pltpu.sync_copy(dst_hbm.at[wid, g], didx_all)
            fire(0, 0)
            lax.fori_loop(0, (idxc + 1) // 2, pair, 0)
        plsc.subcore_barrier()

        def dump(c0):
            pltpu.async_copy(accum.at[pl.ds(c0 * CH, CH)],
                             out_hbm.at[cid, pl.ds(c0 * CH, CH)], gsem0)

        def dump_wait(c0):
            pltpu.make_async_copy(
                accum.at[pl.ds(c0 * CH, CH)],
                out_hbm.at[cid, pl.ds(c0 * CH, CH)], gsem0).wait()

        for t in range(cpt):
            dump(sid * cpt + t)
        if rem:
            @pl.when(sid < rem)
            def _dump_extra():
                dump(cpt * ns + sid)
        for t in range(cpt):
            dump_wait(sid * cpt + t)
        if rem:
            @pl.when(sid < rem)
            def _dump_wait_extra():
                dump_wait(cpt * ns + sid)

    return k


_SC_INFO = plsc.get_sparse_core_info()
_NW = _SC_INFO.num_cores * _SC_INFO.num_subcores

_B1, _IDXC1 = 80, 125
_B2, _IDXC2 = 40, 50
_sc_layer1 = _make_sc_edge_kernel(_N, _E, 64, 8, _B1, _IDXC1)
_sc_layer2 = _make_sc_edge_kernel(_N, _E, 128, 1, _B2, _IDXC2, bf16_src=True)


def _block_att(att, heads, ch):
    a = att.reshape(heads, ch)
    eye_h = jnp.eye(heads, dtype=jnp.float32)
    return (a[:, :, None] * eye_h[:, None, :]).reshape(heads * ch, heads)


def kernel(x, edge_index, W1, att_src1, att_dst1, b1, W2, att_src2,
           att_dst2, b2):
    f32 = jnp.float32
    src = edge_index[0]
    dst = edge_index[1]

    asrc1 = _block_att(att_src1, 8, 8)
    adst1 = _block_att(att_dst1, 8, 8)
    z64_8 = jnp.zeros((64, 8), f32)
    G1 = jnp.concatenate([jnp.eye(64, dtype=f32), asrc1, z64_8], axis=1)
    Gd1 = jnp.concatenate([adst1, z64_8], axis=1)

    P2 = np.zeros((128, 128), np.float32)
    for c in range(128):
        j, i, par = c // 32, c % 16, (c // 16) % 2
        P2[c, 32 * j + 2 * i + par] = 1.0
    G2 = jnp.concatenate([jnp.asarray(P2), att_src2.reshape(128, 1),
                          jnp.zeros((128, 31), f32)], axis=1)
    Gd2 = jnp.concatenate([att_dst2.reshape(128, 1),
                           jnp.zeros((128, 15), f32)], axis=1)

    S1 = np.zeros((80, 64), np.float32)
    S1[:64, :64] = np.eye(64)
    R1 = np.zeros((80, 64), np.float32)
    for h in range(8):
        R1[64 + h, h * 8:(h + 1) * 8] = 1.0
    S2 = np.zeros((144, 128), np.float32)
    S2[:128, :128] = np.eye(128)
    R2 = np.zeros((144, 128), np.float32)
    R2[128, :] = 1.0
    S1, R1, S2, R2 = map(jnp.asarray, (S1, R1, S2, R2))

    b1r = b1.reshape(1, 64)
    b2r = b2.reshape(1, 128)

    RB = 1000
    G = _N // RB

    src_tab, dst_tab = pl.pallas_call(
        _tc1_body,
        grid=(G,),
        in_specs=[
            pl.BlockSpec((RB, 128), lambda i: (i, 0)),
            pl.BlockSpec((128, 64), lambda i: (0, 0)),
            pl.BlockSpec((64, 80), lambda i: (0, 0)),
            pl.BlockSpec((64, 16), lambda i: (0, 0)),
        ],
        out_specs=[
            pl.BlockSpec((RB, 80), lambda i: (i, 0)),
            pl.BlockSpec((RB, 16), lambda i: (i, 0)),
        ],
        out_shape=[
            jax.ShapeDtypeStruct((_N, 80), f32),
            jax.ShapeDtypeStruct((_N, 16), f32),
        ],
    )(x, W1, G1, Gd1)

    src1 = src.reshape(_NW, -1, _IDXC1, _B1)
    dst1 = dst.reshape(_NW, -1, _IDXC1, _B1)
    acc1 = _sc_layer1(src1, dst1, src_tab, dst_tab)

    src_tab2, dst_tab2 = pl.pallas_call(
        _tc2_body,
        grid=(G,),
        in_specs=[
            pl.BlockSpec((RB, 80), lambda i: (i, 0)),
            pl.BlockSpec((RB, 80), lambda i: (i, 0)),
            pl.BlockSpec((80, 64), lambda i: (0, 0)),
            pl.BlockSpec((80, 64), lambda i: (0, 0)),
            pl.BlockSpec((1, 64), lambda i: (0, 0)),
            pl.BlockSpec((64, 128), lambda i: (0, 0)),
            pl.BlockSpec((128, 160), lambda i: (0, 0)),
            pl.BlockSpec((128, 16), lambda i: (0, 0)),
        ],
        out_specs=[
            pl.BlockSpec((RB, 160), lambda i: (i, 0)),
            pl.BlockSpec((RB, 16), lambda i: (i, 0)),
        ],
        out_shape=[
            jax.ShapeDtypeStruct((_N, 160), f32),
            jax.ShapeDtypeStruct((_N, 16), f32),
        ],
    )(acc1[0], acc1[1], S1, R1, b1r, W2, G2, Gd2)

    src2 = src.reshape(_NW, -1, _IDXC2, _B2)
    dst2 = dst.reshape(_NW, -1, _IDXC2, _B2)
    q2 = jnp.clip(jnp.round(src_tab2 * 512.0), -32768.0,
                  32767.0).astype(jnp.int32)
    src_tab2i = jnp.bitwise_and(q2[:, 0::2], 0xFFFF) | jnp.left_shift(
        q2[:, 1::2], 16)
    acc2 = _sc_layer2(src2, dst2, src_tab2i, dst_tab2)

    out = pl.pallas_call(
        _tc3_body,
        grid=(G,),
        in_specs=[
            pl.BlockSpec((RB, 144), lambda i: (i, 0)),
            pl.BlockSpec((RB, 144), lambda i: (i, 0)),
            pl.BlockSpec((144, 128), lambda i: (0, 0)),
            pl.BlockSpec((144, 128), lambda i: (0, 0)),
            pl.BlockSpec((1, 128), lambda i: (0, 0)),
        ],
        out_specs=pl.BlockSpec((RB, 128), lambda i: (i, 0)),
        out_shape=jax.ShapeDtypeStruct((_N, 128), f32),
    )(acc2[0], acc2[1], S2, R2, b2r)

    return out


# speedup vs baseline: 1.4607x; 1.4607x over previous
import functools
import numpy as np
import jax
import jax.numpy as jnp
from jax import lax
from jax.experimental import pallas as pl
from jax.experimental.pallas import tpu as pltpu
from jax.experimental.pallas import tpu_sc as plsc

_N = 10000
_E = 320000


def _tc1_body(x_ref, w_ref, g_ref, gd_ref, src_ref, dst_ref):
    h = jnp.dot(x_ref[...], w_ref[...], preferred_element_type=jnp.float32)
    src_ref[...] = jnp.dot(h, g_ref[...], preferred_element_type=jnp.float32)
    dst_ref[...] = jnp.dot(h, gd_ref[...], preferred_element_type=jnp.float32)


def _tc2_body(a0_ref, a1_ref, s_ref, r_ref, b_ref, w2_ref, g2_ref, gd2_ref,
              src2_ref, dst2_ref):
    acc = a0_ref[...] + a1_ref[...]
    numer = jnp.dot(acc, s_ref[...], preferred_element_type=jnp.float32)
    denom = jnp.dot(acc, r_ref[...], preferred_element_type=jnp.float32)
    o = numer / (denom + 1e-16) + b_ref[...]
    o = jnp.where(o > 0, o, jnp.exp(o) - 1.0)
    h2 = jnp.dot(o, w2_ref[...], preferred_element_type=jnp.float32)
    src2_ref[...] = jnp.dot(h2, g2_ref[...], preferred_element_type=jnp.float32)
    dst2_ref[...] = jnp.dot(h2, gd2_ref[...], preferred_element_type=jnp.float32)


def _tc3_body(a0_ref, a1_ref, s_ref, r_ref, b_ref, out_ref):
    acc = a0_ref[...] + a1_ref[...]
    numer = jnp.dot(acc, s_ref[...], preferred_element_type=jnp.float32)
    denom = jnp.dot(acc, r_ref[...], preferred_element_type=jnp.float32)
    z = numer / (denom + 1e-16) + b_ref[...]
    m = jnp.max(z, axis=1, keepdims=True)
    out_ref[...] = z - (m + jnp.log(jnp.sum(jnp.exp(z - m), axis=1,
                                            keepdims=True)))


def _make_sc_edge_kernel(n, e, dh, heads, B, idxc):
    drep = 16
    row = dh + drep
    info = plsc.get_sparse_core_info()
    nc, ns = info.num_cores, info.num_subcores
    nw = nc * ns
    epw = e // nw
    nchunks = epw // B
    ngroups = nchunks // idxc
    CH = 40
    nch = n // CH
    cpt = nch // ns
    rem = nch - cpt * ns
    assert epw * nw == e and nchunks * B == epw and nch * CH == n
    assert ngroups * idxc == nchunks and (idxc % 2 == 0 or ngroups == 1)
    mesh = plsc.VectorSubcoreMesh(core_axis_name="c", subcore_axis_name="s")

    @functools.partial(
        pl.kernel,
        out_type=jax.ShapeDtypeStruct((nc, n, row), jnp.float32),
        mesh=mesh,
        compiler_params=pltpu.CompilerParams(use_tc_tiling_on_sc=False),
        scratch_types=[
            pltpu.VMEM((idxc, B), jnp.int32),
            pltpu.VMEM((idxc, B), jnp.int32),
            pltpu.VMEM((B, row), jnp.float32),
            pltpu.VMEM((B, row), jnp.float32),
            pltpu.VMEM((B, drep), jnp.float32),
            pltpu.VMEM((B, drep), jnp.float32),
            pltpu.VMEM((B, row), jnp.float32),
            pltpu.VMEM((B, row), jnp.float32),
            pltpu.VMEM((CH, row), jnp.float32),
            pltpu.VMEM_SHARED((n, row), jnp.float32),
            pltpu.SemaphoreType.DMA,
            pltpu.SemaphoreType.DMA,
        ],
    )
    def k(src_hbm, dst_hbm, stab_hbm, dtab_hbm, out_hbm,
          sidx_all, didx_all,
          rows0, rows1, drows0, drows1, orows0, orows1,
          zbuf, accum, gsem0, gsem1):
        rows = (rows0, rows1)
        drows = (drows0, drows1)
        orows = (orows0, orows1)
        gsem = (gsem0, gsem1)
        cid = lax.axis_index("c")
        sid = lax.axis_index("s")
        wid = sid * nc + cid
        zero = jnp.zeros((16,), jnp.float32)

        def zrow(i, carry):
            for t in range(row // 16):
                zbuf[i, pl.ds(16 * t, 16)] = zero
            return carry

        lax.fori_loop(0, CH, zrow, 0)

        def zcopy(c0):
            pltpu.async_copy(zbuf, accum.at[pl.ds(c0 * CH, CH)], gsem1)

        def zwait(c0):
            pltpu.make_async_copy(zbuf, accum.at[pl.ds(c0 * CH, CH)],
                                  gsem1).wait()

        for t in range(cpt):
            zcopy(sid * cpt + t)
        if rem:
            @pl.when(sid < rem)
            def _zero_extra():
                zcopy(cpt * ns + sid)
        for t in range(cpt):
            zwait(sid * cpt + t)
        if rem:
            @pl.when(sid < rem)
            def _zero_wait_extra():
                zwait(cpt * ns + sid)
        plsc.subcore_barrier()

        def fire(lj, b):
            pltpu.async_copy(stab_hbm.at[sidx_all.at[lj]], rows[b], gsem[b])
            pltpu.async_copy(dtab_hbm.at[didx_all.at[lj]], drows[b], gsem[b])

        def wait_gather(b):
            pltpu.make_async_copy(stab_hbm.at[sidx_all.at[0]], rows[b],
                                  gsem[b]).wait()
            pltpu.make_async_copy(dtab_hbm.at[didx_all.at[0]], drows[b],
                                  gsem[b]).wait()

        def compute(b):
            ro, dro, oro = rows[b], drows[b], orows[b]

            @plsc.parallel_loop(0, B, unroll=4)
            def _edges(ei):
                vas = ro[ei, pl.ds(dh, 16)]
                vad = dro[ei, pl.ds(0, 16)]
                a = vas + vad
                a = jnp.where(a >= 0, a, 0.2 * a)
                p = jnp.exp(a)
                oro[ei, pl.ds(dh, 16)] = p
                if heads == 1:
                    sv0 = p.at[jnp.zeros((16,), jnp.int32)].get(
                        mode='promise_in_bounds')
                for kk in range(dh // 16):
                    if heads == 1:
                        sv = sv0
                    else:
                        pidx = 2 * kk + lax.shift_right_logical(
                            lax.iota(jnp.int32, 16), 3)
                        sv = p.at[pidx].get(mode='promise_in_bounds')
                    oro[ei, pl.ds(16 * kk, 16)] = (
                        ro[ei, pl.ds(16 * kk, 16)] * sv)

        def pair(jj, carry):
            j0 = jj * 2
            for b in range(2):
                lj = j0 + b
                nb = 1 - b

                @pl.when(lj < idxc)
                def _section():
                    @pl.when(lj + 1 < idxc)
                    def _fire_next():
                        fire(lj + 1, nb)

                    wait_gather(b)
                    compute(b)
                    pltpu.sync_copy(orows[b], accum.at[didx_all.at[lj]],
                                    add=True)
            return carry

        for g in range(ngroups):
            pltpu.sync_copy(src_hbm.at[wid, g], sidx_all)
            pltpu.sync_copy(dst_hbm.at[wid, g], didx_all)
            fire(0, 0)
            lax.fori_loop(0, (idxc + 1) // 2, pair, 0)
        plsc.subcore_barrier()

        def dump(c0):
            pltpu.async_copy(accum.at[pl.ds(c0 * CH, CH)],
                             out_hbm.at[cid, pl.ds(c0 * CH, CH)], gsem0)

        def dump_wait(c0):
            pltpu.make_async_copy(
                accum.at[pl.ds(c0 * CH, CH)],
                out_hbm.at[cid, pl.ds(c0 * CH, CH)], gsem0).wait()

        for t in range(cpt):
            dump(sid * cpt + t)
        if rem:
            @pl.when(sid < rem)
            def _dump_extra():
                dump(cpt * ns + sid)
        for t in range(cpt):
            dump_wait(sid * cpt + t)
        if rem:
            @pl.when(sid < rem)
            def _dump_wait_extra():
                dump_wait(cpt * ns + sid)

    return k


_SC_INFO = plsc.get_sparse_core_info()
_NW = _SC_INFO.num_cores * _SC_INFO.num_subcores

_B1, _IDXC1 = 80, 125
_B2, _IDXC2 = 40, 50
_sc_layer1 = _make_sc_edge_kernel(_N, _E, 64, 8, _B1, _IDXC1)
_sc_layer2 = _make_sc_edge_kernel(_N, _E, 128, 1, _B2, _IDXC2)


def _block_att(att, heads, ch):
    a = att.reshape(heads, ch)
    eye_h = jnp.eye(heads, dtype=jnp.float32)
    return (a[:, :, None] * eye_h[:, None, :]).reshape(heads * ch, heads)


def kernel(x, edge_index, W1, att_src1, att_dst1, b1, W2, att_src2,
           att_dst2, b2):
    f32 = jnp.float32
    src = edge_index[0]
    dst = edge_index[1]

    asrc1 = _block_att(att_src1, 8, 8)
    adst1 = _block_att(att_dst1, 8, 8)
    z64_8 = jnp.zeros((64, 8), f32)
    G1 = jnp.concatenate([jnp.eye(64, dtype=f32), asrc1, z64_8], axis=1)
    Gd1 = jnp.concatenate([adst1, z64_8], axis=1)

    z128_15 = jnp.zeros((128, 15), f32)
    G2 = jnp.concatenate([jnp.eye(128, dtype=f32), att_src2.reshape(128, 1),
                          z128_15], axis=1)
    Gd2 = jnp.concatenate([att_dst2.reshape(128, 1), z128_15], axis=1)

    S1 = np.zeros((80, 64), np.float32)
    S1[:64, :64] = np.eye(64)
    R1 = np.zeros((80, 64), np.float32)
    for h in range(8):
        R1[64 + h, h * 8:(h + 1) * 8] = 1.0
    S2 = np.zeros((144, 128), np.float32)
    S2[:128, :128] = np.eye(128)
    R2 = np.zeros((144, 128), np.float32)
    R2[128, :] = 1.0
    S1, R1, S2, R2 = map(jnp.asarray, (S1, R1, S2, R2))

    b1r = b1.reshape(1, 64)
    b2r = b2.reshape(1, 128)

    RB = 1000
    G = _N // RB

    src_tab, dst_tab = pl.pallas_call(
        _tc1_body,
        grid=(G,),
        in_specs=[
            pl.BlockSpec((RB, 128), lambda i: (i, 0)),
            pl.BlockSpec((128, 64), lambda i: (0, 0)),
            pl.BlockSpec((64, 80), lambda i: (0, 0)),
            pl.BlockSpec((64, 16), lambda i: (0, 0)),
        ],
        out_specs=[
            pl.BlockSpec((RB, 80), lambda i: (i, 0)),
            pl.BlockSpec((RB, 16), lambda i: (i, 0)),
        ],
        out_shape=[
            jax.ShapeDtypeStruct((_N, 80), f32),
            jax.ShapeDtypeStruct((_N, 16), f32),
        ],
    )(x, W1, G1, Gd1)

    src1 = src.reshape(_NW, -1, _IDXC1, _B1)
    dst1 = dst.reshape(_NW, -1, _IDXC1, _B1)
    acc1 = _sc_layer1(src1, dst1, src_tab, dst_tab)

    src_tab2, dst_tab2 = pl.pallas_call(
        _tc2_body,
        grid=(G,),
        in_specs=[
            pl.BlockSpec((RB, 80), lambda i: (i, 0)),
            pl.BlockSpec((RB, 80), lambda i: (i, 0)),
            pl.BlockSpec((80, 64), lambda i: (0, 0)),
            pl.BlockSpec((80, 64), lambda i: (0, 0)),
            pl.BlockSpec((1, 64), lambda i: (0, 0)),
            pl.BlockSpec((64, 128), lambda i: (0, 0)),
            pl.BlockSpec((128, 144), lambda i: (0, 0)),
            pl.BlockSpec((128, 16), lambda i: (0, 0)),
        ],
        out_specs=[
            pl.BlockSpec((RB, 144), lambda i: (i, 0)),
            pl.BlockSpec((RB, 16), lambda i: (i, 0)),
        ],
        out_shape=[
            jax.ShapeDtypeStruct((_N, 144), f32),
            jax.ShapeDtypeStruct((_N, 16), f32),
        ],
    )(acc1[0], acc1[1], S1, R1, b1r, W2, G2, Gd2)

    src2 = src.reshape(_NW, -1, _IDXC2, _B2)
    dst2 = dst.reshape(_NW, -1, _IDXC2, _B2)
    acc2 = _sc_layer2(src2, dst2, src_tab2, dst_tab2)

    out = pl.pallas_call(
        _tc3_body,
        grid=(G,),
        in_specs=[
            pl.BlockSpec((RB, 144), lambda i: (i, 0)),
            pl.BlockSpec((RB, 144), lambda i: (i, 0)),
            pl.BlockSpec((144, 128), lambda i: (0, 0)),
            pl.BlockSpec((144, 128), lambda i: (0, 0)),
            pl.BlockSpec((1, 128), lambda i: (0, 0)),
        ],
        out_specs=pl.BlockSpec((RB, 128), lambda i: (i, 0)),
        out_shape=jax.ShapeDtypeStruct((_N, 128), f32),
    )(acc2[0], acc2[1], S2, R2, b2r)

    return out


# speedup vs baseline: 1.4675x; 1.0047x over previous
import functools
import numpy as np
import jax
import jax.numpy as jnp
from jax import lax
from jax.experimental import pallas as pl
from jax.experimental.pallas import tpu as pltpu
from jax.experimental.pallas import tpu_sc as plsc

_N = 10000
_E = 320000


def _tc1_body(x_ref, w_ref, g_ref, gd_ref, src_ref, dst_ref):
    h = jnp.dot(x_ref[...], w_ref[...], preferred_element_type=jnp.float32)
    src_ref[...] = jnp.dot(h, g_ref[...], preferred_element_type=jnp.float32)
    dst_ref[...] = jnp.dot(h, gd_ref[...], preferred_element_type=jnp.float32)


def _tc2_body(a0_ref, a1_ref, s_ref, r_ref, b_ref, w2_ref, g2_ref, gd2_ref,
              src2_ref, dst2_ref):
    acc = a0_ref[...] + a1_ref[...]
    numer = jnp.dot(acc, s_ref[...], preferred_element_type=jnp.float32)
    denom = jnp.dot(acc, r_ref[...], preferred_element_type=jnp.float32)
    o = numer / (denom + 1e-16) + b_ref[...]
    o = jnp.where(o > 0, o, jnp.exp(o) - 1.0)
    h2 = jnp.dot(o, w2_ref[...], preferred_element_type=jnp.float32)
    src2_ref[...] = jnp.dot(h2, g2_ref[...], preferred_element_type=jnp.float32)
    dst2_ref[...] = jnp.dot(h2, gd2_ref[...], preferred_element_type=jnp.float32)


def _tc3_body(a0_ref, a1_ref, s_ref, r_ref, b_ref, out_ref):
    acc = a0_ref[...] + a1_ref[...]
    numer = jnp.dot(acc, s_ref[...], preferred_element_type=jnp.float32)
    denom = jnp.dot(acc, r_ref[...], preferred_element_type=jnp.float32)
    z = numer / (denom + 1e-16) + b_ref[...]
    m = jnp.max(z, axis=1, keepdims=True)
    out_ref[...] = z - (m + jnp.log(jnp.sum(jnp.exp(z - m), axis=1,
                                            keepdims=True)))


def _make_sc_edge_kernel(n, e, dh, heads, B, idxc):
    drep = 16
    row = dh + drep
    info = plsc.get_sparse_core_info()
    nc, ns = info.num_cores, info.num_subcores
    nw = nc * ns
    epw = e // nw
    nchunks = epw // B
    ngroups = nchunks // idxc
    CH = 40
    nch = n // CH
    cpt = nch // ns
    rem = nch - cpt * ns
    assert epw * nw == e and nchunks * B == epw and nch * CH == n
    assert ngroups * idxc == nchunks and (idxc % 2 == 0 or ngroups == 1)
    mesh = plsc.VectorSubcoreMesh(core_axis_name="c", subcore_axis_name="s")

    @functools.partial(
        pl.kernel,
        out_type=jax.ShapeDtypeStruct((nc, n, row), jnp.float32),
        mesh=mesh,
        compiler_params=pltpu.CompilerParams(use_tc_tiling_on_sc=False),
        scratch_types=[
            pltpu.VMEM((idxc, B), jnp.int32),
            pltpu.VMEM((idxc, B), jnp.int32),
            pltpu.VMEM((B, row), jnp.float32),
            pltpu.VMEM((B, row), jnp.float32),
            pltpu.VMEM((B, drep), jnp.float32),
            pltpu.VMEM((B, drep), jnp.float32),
            pltpu.VMEM((B, row), jnp.float32),
            pltpu.VMEM((B, row), jnp.float32),
            pltpu.VMEM((CH, row), jnp.float32),
            pltpu.VMEM_SHARED((n, row), jnp.float32),
            pltpu.SemaphoreType.DMA,
            pltpu.SemaphoreType.DMA,
        ],
    )
    def k(src_hbm, dst_hbm, stab_hbm, dtab_hbm, out_hbm,
          sidx_all, didx_all,
          rows0, rows1, drows0, drows1, orows0, orows1,
          zbuf, accum, gsem0, gsem1):
        rows = (rows0, rows1)
        drows = (drows0, drows1)
        orows = (orows0, orows1)
        gsem = (gsem0, gsem1)
        cid = lax.axis_index("c")
        sid = lax.axis_index("s")
        wid = sid * nc + cid
        zero = jnp.zeros((16,), jnp.float32)

        def zrow(i, carry):
            for t in range(row // 16):
                zbuf[i, pl.ds(16 * t, 16)] = zero
            return carry

        lax.fori_loop(0, CH, zrow, 0)

        def zcopy(c0):
            pltpu.async_copy(zbuf, accum.at[pl.ds(c0 * CH, CH)], gsem1)

        def zwait(c0):
            pltpu.make_async_copy(zbuf, accum.at[pl.ds(c0 * CH, CH)],
                                  gsem1).wait()

        for t in range(cpt):
            zcopy(sid * cpt + t)
        if rem:
            @pl.when(sid < rem)
            def _zero_extra():
                zcopy(cpt * ns + sid)
        for t in range(cpt):
            zwait(sid * cpt + t)
        if rem:
            @pl.when(sid < rem)
            def _zero_wait_extra():
                zwait(cpt * ns + sid)
        plsc.subcore_barrier()

        def fire(lj, b):
            pltpu.async_copy(stab_hbm.at[sidx_all.at[lj]], rows[b], gsem[b])
            pltpu.async_copy(dtab_hbm.at[didx_all.at[lj]], drows[b], gsem[b])

        def wait_gather(b):
            pltpu.make_async_copy(stab_hbm.at[sidx_all.at[0]], rows[b],
                                  gsem[b]).wait()
            pltpu.make_async_copy(dtab_hbm.at[didx_all.at[0]], drows[b],
                                  gsem[b]).wait()

        def compute(b):
            ro, dro, oro = rows[b], drows[b], orows[b]

            @plsc.parallel_loop(0, B, unroll=8)
            def _edges(ei):
                vas = ro[ei, pl.ds(dh, 16)]
                vad = dro[ei, pl.ds(0, 16)]
                a = vas + vad
                a = jnp.where(a >= 0, a, 0.2 * a)
                p = jnp.exp(a)
                oro[ei, pl.ds(dh, 16)] = p
                if heads == 1:
                    sv0 = p.at[jnp.zeros((16,), jnp.int32)].get(
                        mode='promise_in_bounds')
                for kk in range(dh // 16):
                    if heads == 1:
                        sv = sv0
                    else:
                        pidx = 2 * kk + lax.shift_right_logical(
                            lax.iota(jnp.int32, 16), 3)
                        sv = p.at[pidx].get(mode='promise_in_bounds')
                    oro[ei, pl.ds(16 * kk, 16)] = (
                        ro[ei, pl.ds(16 * kk, 16)] * sv)

        def pair(jj, carry):
            j0 = jj * 2
            for b in range(2):
                lj = j0 + b
                nb = 1 - b

                @pl.when(lj < idxc)
                def _section():
                    @pl.when(lj + 1 < idxc)
                    def _fire_next():
                        fire(lj + 1, nb)

                    wait_gather(b)
                    compute(b)
                    pltpu.sync_copy(orows[b], accum.at[didx_all.at[lj]],
                                    add=True)
            return carry

        for g in range(ngroups):
            pltpu.sync_copy(src_hbm.at[wid, g], sidx_all)
            pltpu.sync_copy(dst_hbm.at[wid, g], didx_all)
            fire(0, 0)
            lax.fori_loop(0, (idxc + 1) // 2, pair, 0)
        plsc.subcore_barrier()

        def dump(c0):
            pltpu.async_copy(accum.at[pl.ds(c0 * CH, CH)],
                             out_hbm.at[cid, pl.ds(c0 * CH, CH)], gsem0)

        def dump_wait(c0):
            pltpu.make_async_copy(
                accum.at[pl.ds(c0 * CH, CH)],
                out_hbm.at[cid, pl.ds(c0 * CH, CH)], gsem0).wait()

        for t in range(cpt):
            dump(sid * cpt + t)
        if rem:
            @pl.when(sid < rem)
            def _dump_extra():
                dump(cpt * ns + sid)
        for t in range(cpt):
            dump_wait(sid * cpt + t)
        if rem:
            @pl.when(sid < rem)
            def _dump_wait_extra():
                dump_wait(cpt * ns + sid)

    return k


_SC_INFO = plsc.get_sparse_core_info()
_NW = _SC_INFO.num_cores * _SC_INFO.num_subcores

_B1, _IDXC1 = 80, 125
_B2, _IDXC2 = 40, 50
_sc_layer1 = _make_sc_edge_kernel(_N, _E, 64, 8, _B1, _IDXC1)
_sc_layer2 = _make_sc_edge_kernel(_N, _E, 128, 1, _B2, _IDXC2)


def _block_att(att, heads, ch):
    a = att.reshape(heads, ch)
    eye_h = jnp.eye(heads, dtype=jnp.float32)
    return (a[:, :, None] * eye_h[:, None, :]).reshape(heads * ch, heads)


def kernel(x, edge_index, W1, att_src1, att_dst1, b1, W2, att_src2,
           att_dst2, b2):
    f32 = jnp.float32
    src = edge_index[0]
    dst = edge_index[1]

    asrc1 = _block_att(att_src1, 8, 8)
    adst1 = _block_att(att_dst1, 8, 8)
    z64_8 = jnp.zeros((64, 8), f32)
    G1 = jnp.concatenate([jnp.eye(64, dtype=f32), asrc1, z64_8], axis=1)
    Gd1 = jnp.concatenate([adst1, z64_8], axis=1)

    z128_15 = jnp.zeros((128, 15), f32)
    G2 = jnp.concatenate([jnp.eye(128, dtype=f32), att_src2.reshape(128, 1),
                          z128_15], axis=1)
    Gd2 = jnp.concatenate([att_dst2.reshape(128, 1), z128_15], axis=1)

    S1 = np.zeros((80, 64), np.float32)
    S1[:64, :64] = np.eye(64)
    R1 = np.zeros((80, 64), np.float32)
    for h in range(8):
        R1[64 + h, h * 8:(h + 1) * 8] = 1.0
    S2 = np.zeros((144, 128), np.float32)
    S2[:128, :128] = np.eye(128)
    R2 = np.zeros((144, 128), np.float32)
    R2[128, :] = 1.0
    S1, R1, S2, R2 = map(jnp.asarray, (S1, R1, S2, R2))

    b1r = b1.reshape(1, 64)
    b2r = b2.reshape(1, 128)

    RB = 1000
    G = _N // RB

    src_tab, dst_tab = pl.pallas_call(
        _tc1_body,
        grid=(G,),
        in_specs=[
            pl.BlockSpec((RB, 128), lambda i: (i, 0)),
            pl.BlockSpec((128, 64), lambda i: (0, 0)),
            pl.BlockSpec((64, 80), lambda i: (0, 0)),
            pl.BlockSpec((64, 16), lambda i: (0, 0)),
        ],
        out_specs=[
            pl.BlockSpec((RB, 80), lambda i: (i, 0)),
            pl.BlockSpec((RB, 16), lambda i: (i, 0)),
        ],
        out_shape=[
            jax.ShapeDtypeStruct((_N, 80), f32),
            jax.ShapeDtypeStruct((_N, 16), f32),
        ],
    )(x, W1, G1, Gd1)

    src1 = src.reshape(_NW, -1, _IDXC1, _B1)
    dst1 = dst.reshape(_NW, -1, _IDXC1, _B1)
    acc1 = _sc_layer1(src1, dst1, src_tab, dst_tab)

    src_tab2, dst_tab2 = pl.pallas_call(
        _tc2_body,
        grid=(G,),
        in_specs=[
            pl.BlockSpec((RB, 80), lambda i: (i, 0)),
            pl.BlockSpec((RB, 80), lambda i: (i, 0)),
            pl.BlockSpec((80, 64), lambda i: (0, 0)),
            pl.BlockSpec((80, 64), lambda i: (0, 0)),
            pl.BlockSpec((1, 64), lambda i: (0, 0)),
            pl.BlockSpec((64, 128), lambda i: (0, 0)),
            pl.BlockSpec((128, 144), lambda i: (0, 0)),
            pl.BlockSpec((128, 16), lambda i: (0, 0)),
        ],
        out_specs=[
            pl.BlockSpec((RB, 144), lambda i: (i, 0)),
            pl.BlockSpec((RB, 16), lambda i: (i, 0)),
        ],
        out_shape=[
            jax.ShapeDtypeStruct((_N, 144), f32),
            jax.ShapeDtypeStruct((_N, 16), f32),
        ],
    )(acc1[0], acc1[1], S1, R1, b1r, W2, G2, Gd2)

    src2 = src.reshape(_NW, -1, _IDXC2, _B2)
    dst2 = dst.reshape(_NW, -1, _IDXC2, _B2)
    acc2 = _sc_layer2(src2, dst2, src_tab2, dst_tab2)

    out = pl.pallas_call(
        _tc3_body,
        grid=(G,),
        in_specs=[
            pl.BlockSpec((RB, 144), lambda i: (i, 0)),
            pl.BlockSpec((RB, 144), lambda i: (i, 0)),
            pl.BlockSpec((144, 128), lambda i: (0, 0)),
            pl.BlockSpec((144, 128), lambda i: (0, 0)),
            pl.BlockSpec((1, 128), lambda i: (0, 0)),
        ],
        out_specs=pl.BlockSpec((RB, 128), lambda i: (i, 0)),
        out_shape=jax.ShapeDtypeStruct((_N, 128), f32),
    )(acc2[0], acc2[1], S2, R2, b2r)

    return out


# speedup vs baseline: 1.4954x; 1.0190x over previous
import functools
import numpy as np
import jax
import jax.numpy as jnp
from jax import lax
from jax.experimental import pallas as pl
from jax.experimental.pallas import tpu as pltpu
from jax.experimental.pallas import tpu_sc as plsc

_N = 10000
_E = 320000


def _tc1_body(x_ref, w_ref, g_ref, gd_ref, src_ref, dst_ref):
    h = jnp.dot(x_ref[...], w_ref[...], preferred_element_type=jnp.float32)
    src_ref[...] = jnp.dot(h, g_ref[...], preferred_element_type=jnp.float32)
    dst_ref[...] = jnp.dot(h, gd_ref[...], preferred_element_type=jnp.float32)


def _tc2_body(a0_ref, a1_ref, s_ref, r_ref, b_ref, w2_ref, g2_ref, gd2_ref,
              src2_ref, dst2_ref):
    acc = a0_ref[...] + a1_ref[...]
    numer = jnp.dot(acc, s_ref[...], preferred_element_type=jnp.float32)
    denom = jnp.dot(acc, r_ref[...], preferred_element_type=jnp.float32)
    o = numer / (denom + 1e-16) + b_ref[...]
    o = jnp.where(o > 0, o, jnp.exp(o) - 1.0)
    h2 = jnp.dot(o, w2_ref[...], preferred_element_type=jnp.float32)
    src2_ref[...] = jnp.dot(h2, g2_ref[...], preferred_element_type=jnp.float32)
    dst2_ref[...] = jnp.dot(h2, gd2_ref[...], preferred_element_type=jnp.float32)


def _tc3_body(a0_ref, a1_ref, s_ref, r_ref, b_ref, out_ref):
    acc = a0_ref[...] + a1_ref[...]
    numer = jnp.dot(acc, s_ref[...], preferred_element_type=jnp.float32)
    denom = jnp.dot(acc, r_ref[...], preferred_element_type=jnp.float32)
    z = numer / (denom + 1e-16) + b_ref[...]
    m = jnp.max(z, axis=1, keepdims=True)
    out_ref[...] = z - (m + jnp.log(jnp.sum(jnp.exp(z - m), axis=1,
                                            keepdims=True)))


def _make_sc_edge_kernel(n, e, dh, heads, B, idxc):
    drep = 16
    row = dh + drep
    info = plsc.get_sparse_core_info()
    nc, ns = info.num_cores, info.num_subcores
    nw = nc * ns
    epw = e // nw
    nchunks = epw // B
    ngroups = nchunks // idxc
    CH = 40
    nch = n // CH
    cpt = nch // ns
    rem = nch - cpt * ns
    assert epw * nw == e and nchunks * B == epw and nch * CH == n
    assert ngroups * idxc == nchunks and (idxc % 2 == 0 or ngroups == 1)
    mesh = plsc.VectorSubcoreMesh(core_axis_name="c", subcore_axis_name="s")

    @functools.partial(
        pl.kernel,
        out_type=jax.ShapeDtypeStruct((nc, n, row), jnp.float32),
        mesh=mesh,
        compiler_params=pltpu.CompilerParams(use_tc_tiling_on_sc=False),
        scratch_types=[
            pltpu.VMEM((idxc, B), jnp.int32),
            pltpu.VMEM((idxc, B), jnp.int32),
            pltpu.VMEM((B, row), jnp.float32),
            pltpu.VMEM((B, row), jnp.float32),
            pltpu.VMEM((B, drep), jnp.float32),
            pltpu.VMEM((B, drep), jnp.float32),
            pltpu.VMEM((B, row), jnp.float32),
            pltpu.VMEM((B, row), jnp.float32),
            pltpu.VMEM((CH, row), jnp.float32),
            pltpu.VMEM_SHARED((n, row), jnp.float32),
            pltpu.SemaphoreType.DMA,
            pltpu.SemaphoreType.DMA,
        ],
    )
    def k(src_hbm, dst_hbm, stab_hbm, dtab_hbm, out_hbm,
          sidx_all, didx_all,
          rows0, rows1, drows0, drows1, orows0, orows1,
          zbuf, accum, gsem0, gsem1):
        rows = (rows0, rows1)
        drows = (drows0, drows1)
        orows = (orows0, orows1)
        gsem = (gsem0, gsem1)
        cid = lax.axis_index("c")
        sid = lax.axis_index("s")
        wid = sid * nc + cid
        zero = jnp.zeros((16,), jnp.float32)

        def zrow(i, carry):
            for t in range(row // 16):
                zbuf[i, pl.ds(16 * t, 16)] = zero
            return carry

        lax.fori_loop(0, CH, zrow, 0)

        def zcopy(c0):
            pltpu.async_copy(zbuf, accum.at[pl.ds(c0 * CH, CH)], gsem1)

        def zwait(c0):
            pltpu.make_async_copy(zbuf, accum.at[pl.ds(c0 * CH, CH)],
                                  gsem1).wait()

        for t in range(cpt):
            zcopy(sid * cpt + t)
        if rem:
            @pl.when(sid < rem)
            def _zero_extra():
                zcopy(cpt * ns + sid)
        for t in range(cpt):
            zwait(sid * cpt + t)
        if rem:
            @pl.when(sid < rem)
            def _zero_wait_extra():
                zwait(cpt * ns + sid)
        plsc.subcore_barrier()

        def fire(lj, b):
            pltpu.async_copy(stab_hbm.at[sidx_all.at[lj]], rows[b], gsem[b])
            pltpu.async_copy(dtab_hbm.at[didx_all.at[lj]], drows[b], gsem[b])

        def wait_gather(b):
            pltpu.make_async_copy(stab_hbm.at[sidx_all.at[0]], rows[b],
                                  gsem[b]).wait()
            pltpu.make_async_copy(dtab_hbm.at[didx_all.at[0]], drows[b],
                                  gsem[b]).wait()

        def compute(b):
            ro, dro, oro = rows[b], drows[b], orows[b]

            @plsc.parallel_loop(0, B, unroll=8)
            def _edges(ei):
                vas = ro[ei, pl.ds(dh, 16)]
                vad = dro[ei, pl.ds(0, 16)]
                a = vas + vad
                a = jnp.where(a >= 0, a, 0.2 * a)
                p = jnp.exp(a)
                oro[ei, pl.ds(dh, 16)] = p
                if heads == 1:
                    sv0 = p.at[jnp.zeros((16,), jnp.int32)].get(
                        mode='promise_in_bounds')
                for kk in range(dh // 16):
                    if heads == 1:
                        sv = sv0
                    else:
                        pidx = 2 * kk + lax.shift_right_logical(
                            lax.iota(jnp.int32, 16), 3)
                        sv = p.at[pidx].get(mode='promise_in_bounds')
                    oro[ei, pl.ds(16 * kk, 16)] = (
                        ro[ei, pl.ds(16 * kk, 16)] * sv)

        def pair(jj, carry):
            j0 = jj * 2
            for b in range(2):
                lj = j0 + b
                nb = 1 - b

                @pl.when(lj < idxc)
                def _section():
                    @pl.when(lj + 1 < idxc)
                    def _fire_next():
                        fire(lj + 1, nb)

                    wait_gather(b)
                    compute(b)
                    pltpu.sync_copy(orows[b], accum.at[didx_all.at[lj]],
                                    add=True)
            return carry

        for g in range(ngroups):
            pltpu.sync_copy(src_hbm.at[wid, g], sidx_all)
            pltpu.sync_copy(dst_hbm.at[wid, g], didx_all)
            fire(0, 0)
            lax.fori_loop(0, (idxc + 1) // 2, pair, 0)
        plsc.subcore_barrier()

        def dump(c0):
            pltpu.async_copy(accum.at[pl.ds(c0 * CH, CH)],
                             out_hbm.at[cid, pl.ds(c0 * CH, CH)], gsem0)

        def dump_wait(c0):
            pltpu.make_async_copy(
                accum.at[pl.ds(c0 * CH, CH)],
                out_hbm.at[cid, pl.ds(c0 * CH, CH)], gsem0).wait()

        for t in range(cpt):
            dump(sid * cpt + t)
        if rem:
            @pl.when(sid < rem)
            def _dump_extra():
                dump(cpt * ns + sid)
        for t in range(cpt):
            dump_wait(sid * cpt + t)
        if rem:
            @pl.when(sid < rem)
            def _dump_wait_extra():
                dump_wait(cpt * ns + sid)

    return k


_SC_INFO = plsc.get_sparse_core_info()
_NW = _SC_INFO.num_cores * _SC_INFO.num_subcores

_B1, _IDXC1 = 80, 125
_B2, _IDXC2 = 40, 50
_sc_layer1 = _make_sc_edge_kernel(_N, _E, 64, 8, _B1, _IDXC1)
_sc_layer2 = _make_sc_edge_kernel(_N, _E, 128, 1, _B2, _IDXC2)


def _block_att(att, heads, ch):
    a = att.reshape(heads, ch)
    eye_h = jnp.eye(heads, dtype=jnp.float32)
    return (a[:, :, None] * eye_h[:, None, :]).reshape(heads * ch, heads)


def kernel(x, edge_index, W1, att_src1, att_dst1, b1, W2, att_src2,
           att_dst2, b2):
    f32 = jnp.float32
    src = edge_index[0]
    dst = edge_index[1]

    asrc1 = _block_att(att_src1, 8, 8)
    adst1 = _block_att(att_dst1, 8, 8)
    z64_8 = jnp.zeros((64, 8), f32)
    G1 = jnp.concatenate([jnp.eye(64, dtype=f32), asrc1, z64_8], axis=1)
    Gd1 = jnp.concatenate([adst1, z64_8], axis=1)

    z128_15 = jnp.zeros((128, 15), f32)
    G2 = jnp.concatenate([jnp.eye(128, dtype=f32), att_src2.reshape(128, 1),
                          z128_15], axis=1)
    Gd2 = jnp.concatenate([att_dst2.reshape(128, 1), z128_15], axis=1)

    S1 = np.zeros((80, 64), np.float32)
    S1[:64, :64] = np.eye(64)
    R1 = np.zeros((80, 64), np.float32)
    for h in range(8):
        R1[64 + h, h * 8:(h + 1) * 8] = 1.0
    S2 = np.zeros((144, 128), np.float32)
    S2[:128, :128] = np.eye(128)
    R2 = np.zeros((144, 128), np.float32)
    R2[128, :] = 1.0
    S1, R1, S2, R2 = map(jnp.asarray, (S1, R1, S2, R2))

    b1r = b1.reshape(1, 64)
    b2r = b2.reshape(1, 128)

    RB = 2000
    G = _N // RB

    src_tab, dst_tab = pl.pallas_call(
        _tc1_body,
        grid=(G,),
        in_specs=[
            pl.BlockSpec((RB, 128), lambda i: (i, 0)),
            pl.BlockSpec((128, 64), lambda i: (0, 0)),
            pl.BlockSpec((64, 80), lambda i: (0, 0)),
            pl.BlockSpec((64, 16), lambda i: (0, 0)),
        ],
        out_specs=[
            pl.BlockSpec((RB, 80), lambda i: (i, 0)),
            pl.BlockSpec((RB, 16), lambda i: (i, 0)),
        ],
        out_shape=[
            jax.ShapeDtypeStruct((_N, 80), f32),
            jax.ShapeDtypeStruct((_N, 16), f32),
        ],
    )(x, W1, G1, Gd1)

    src1 = src.reshape(_NW, -1, _IDXC1, _B1)
    dst1 = dst.reshape(_NW, -1, _IDXC1, _B1)
    acc1 = _sc_layer1(src1, dst1, src_tab, dst_tab)

    src_tab2, dst_tab2 = pl.pallas_call(
        _tc2_body,
        grid=(G,),
        in_specs=[
            pl.BlockSpec((RB, 80), lambda i: (i, 0)),
            pl.BlockSpec((RB, 80), lambda i: (i, 0)),
            pl.BlockSpec((80, 64), lambda i: (0, 0)),
            pl.BlockSpec((80, 64), lambda i: (0, 0)),
            pl.BlockSpec((1, 64), lambda i: (0, 0)),
            pl.BlockSpec((64, 128), lambda i: (0, 0)),
            pl.BlockSpec((128, 144), lambda i: (0, 0)),
            pl.BlockSpec((128, 16), lambda i: (0, 0)),
        ],
        out_specs=[
            pl.BlockSpec((RB, 144), lambda i: (i, 0)),
            pl.BlockSpec((RB, 16), lambda i: (i, 0)),
        ],
        out_shape=[
            jax.ShapeDtypeStruct((_N, 144), f32),
            jax.ShapeDtypeStruct((_N, 16), f32),
        ],
    )(acc1[0], acc1[1], S1, R1, b1r, W2, G2, Gd2)

    src2 = src.reshape(_NW, -1, _IDXC2, _B2)
    dst2 = dst.reshape(_NW, -1, _IDXC2, _B2)
    acc2 = _sc_layer2(src2, dst2, src_tab2, dst_tab2)

    out = pl.pallas_call(
        _tc3_body,
        grid=(G,),
        in_specs=[
            pl.BlockSpec((RB, 144), lambda i: (i, 0)),
            pl.BlockSpec((RB, 144), lambda i: (i, 0)),
            pl.BlockSpec((144, 128), lambda i: (0, 0)),
            pl.BlockSpec((144, 128), lambda i: (0, 0)),
            pl.BlockSpec((1, 128), lambda i: (0, 0)),
        ],
        out_specs=pl.BlockSpec((RB, 128), lambda i: (i, 0)),
        out_shape=jax.ShapeDtypeStruct((_N, 128), f32),
    )(acc2[0], acc2[1], S2, R2, b2r)

    return out
